# initial kernel scaffold (unmeasured)
import jax
import jax.numpy as jnp
from jax import lax
from jax.experimental import pallas as pl
from jax.experimental.pallas import tpu as pltpu

N_DEV = 16
M, K_SH, N = 4096, 256, 8192
CHUNK = M // N_DEV


def kernel(x, w_mat):
    x = x.astype(jnp.bfloat16)
    w = w_mat.astype(jnp.bfloat16)

    def body(x_ref, w_ref, out_ref, comm_ref, send_sems, recv_sems,
             ag_sems, credit_sem):
        d = lax.axis_index("i")
        left = lax.rem(d + (N_DEV - 1), N_DEV)
        right = lax.rem(d + 1, N_DEV)

        barrier_sem = pltpu.get_barrier_semaphore()
        for nbr in (left, right):
            pl.semaphore_signal(
                barrier_sem, inc=1,
                device_id=(nbr,), device_id_type=pl.DeviceIdType.MESH,
            )
        pl.semaphore_wait(barrier_sem, 2)

        def partial_chunk(c):
            xs = x_ref[pl.ds(c * CHUNK, CHUNK), :]
            return jnp.dot(xs, w_ref[:, :], preferred_element_type=jnp.float32)

        for h in range(N_DEV - 1):
            c = lax.rem(d + (2 * N_DEV - 1 - h), N_DEV)
            slot = h % 2
            p = partial_chunk(c)
            if h == 0:
                comm_ref[slot] = p.astype(jnp.bfloat16)
            else:
                acc = comm_ref[slot].astype(jnp.float32) + p
                comm_ref[slot] = acc.astype(jnp.bfloat16)
            if h >= 1:
                pl.semaphore_wait(credit_sem, 1)
            rdma = pltpu.make_async_remote_copy(
                src_ref=comm_ref.at[slot],
                dst_ref=comm_ref.at[(h + 1) % 2],
                send_sem=send_sems.at[slot],
                recv_sem=recv_sems.at[(h + 1) % 2],
                device_id=(right,),
                device_id_type=pl.DeviceIdType.MESH,
            )
            rdma.start()
            rdma.wait()
            if h <= N_DEV - 3:
                pl.semaphore_signal(
                    credit_sem, inc=1,
                    device_id=(left,), device_id_type=pl.DeviceIdType.MESH,
                )

        own = comm_ref[(N_DEV - 1) % 2].astype(jnp.float32) + partial_chunk(d)
        out_ref[pl.ds(d * CHUNK, CHUNK), :] = own.astype(jnp.bfloat16)

        for t in range(N_DEV - 1):
            sc = lax.rem(d + (2 * N_DEV - t), N_DEV)
            rc = lax.rem(d + (2 * N_DEV - 1 - t), N_DEV)
            rdma = pltpu.make_async_remote_copy(
                src_ref=out_ref.at[pl.ds(sc * CHUNK, CHUNK), :],
                dst_ref=out_ref.at[pl.ds(rc * CHUNK, CHUNK), :],
                send_sem=send_sems.at[t % 2],
                recv_sem=ag_sems.at[t],
                device_id=(right,),
                device_id_type=pl.DeviceIdType.MESH,
            )
            rdma.start()
            rdma.wait()

        amax = jnp.float32(0.0)
        for i in range(N_DEV):
            chunk = out_ref[pl.ds(i * CHUNK, CHUNK), :]
            amax = jnp.maximum(amax, jnp.max(jnp.abs(chunk)).astype(jnp.float32))
        scale = amax / 127.0
        inv = 127.0 / amax
        for i in range(N_DEV):
            y = out_ref[pl.ds(i * CHUNK, CHUNK), :].astype(jnp.float32)
            q = jnp.clip(jnp.round(y * inv), -127.0, 127.0)
            out_ref[pl.ds(i * CHUNK, CHUNK), :] = (q * scale).astype(jnp.bfloat16)

    return pl.pallas_call(
        body,
        out_shape=jax.ShapeDtypeStruct((M, N), jnp.bfloat16),
        in_specs=[
            pl.BlockSpec(memory_space=pltpu.VMEM),
            pl.BlockSpec(memory_space=pltpu.VMEM),
        ],
        out_specs=pl.BlockSpec(memory_space=pltpu.VMEM),
        scratch_shapes=[
            pltpu.VMEM((2, CHUNK, N), jnp.bfloat16),
            pltpu.SemaphoreType.DMA((2,)),
            pltpu.SemaphoreType.DMA((2,)),
            pltpu.SemaphoreType.DMA((N_DEV - 1,)),
            pltpu.SemaphoreType.REGULAR,
        ],
        compiler_params=pltpu.CompilerParams(collective_id=0),
    )(x, w)


# baseline (device time: 1521121 ns/iter reference)
import jax
import jax.numpy as jnp
from jax import lax
from jax.experimental import pallas as pl
from jax.experimental.pallas import tpu as pltpu

N_DEV = 16
M, K_SH, N = 4096, 256, 8192
CHUNK = M // N_DEV


def kernel(x, w_mat):
    x = x.astype(jnp.bfloat16)
    w = w_mat.astype(jnp.bfloat16)

    def body(x_ref, w_ref, out_hbm, comm_ref, ychunk_ref, amax_ref,
             send_sems, recv_sems, ag_sems, amax_send_sems, amax_sems,
             copy_sem, credit_sem):
        d = lax.axis_index("i")
        left = lax.rem(d + (N_DEV - 1), N_DEV)
        right = lax.rem(d + 1, N_DEV)

        barrier_sem = pltpu.get_barrier_semaphore()
        for nbr in (left, right):
            pl.semaphore_signal(
                barrier_sem, inc=1,
                device_id=(nbr,), device_id_type=pl.DeviceIdType.MESH,
            )
        pl.semaphore_wait(barrier_sem, 2)

        def partial_chunk(c):
            xs = x_ref[pl.ds(c * CHUNK, CHUNK), :]
            return jnp.dot(xs, w_ref[:, :], preferred_element_type=jnp.float32)

        for h in range(N_DEV - 1):
            c = lax.rem(d + (2 * N_DEV - 1 - h), N_DEV)
            slot = h % 2
            p = partial_chunk(c)
            if h == 0:
                comm_ref[slot] = p.astype(jnp.bfloat16)
            else:
                acc = comm_ref[slot].astype(jnp.float32) + p
                comm_ref[slot] = acc.astype(jnp.bfloat16)
            if h >= 1:
                pl.semaphore_wait(credit_sem, 1)
            rdma = pltpu.make_async_remote_copy(
                src_ref=comm_ref.at[slot],
                dst_ref=comm_ref.at[(h + 1) % 2],
                send_sem=send_sems.at[slot],
                recv_sem=recv_sems.at[(h + 1) % 2],
                device_id=(right,),
                device_id_type=pl.DeviceIdType.MESH,
            )
            rdma.start()
            rdma.wait()
            if h <= N_DEV - 3:
                pl.semaphore_signal(
                    credit_sem, inc=1,
                    device_id=(left,), device_id_type=pl.DeviceIdType.MESH,
                )

        own = comm_ref[(N_DEV - 1) % 2].astype(jnp.float32) + partial_chunk(d)
        ychunk_ref[:, :] = own
        amax_d = jnp.max(jnp.abs(own))
        amax_ref[pl.ds(d, 1), :] = jnp.full((1, 128), amax_d, jnp.float32)

        amax_sends = []
        for off in range(1, N_DEV):
            tgt = lax.rem(d + off, N_DEV)
            sd = pltpu.make_async_remote_copy(
                src_ref=amax_ref.at[pl.ds(d, 1), :],
                dst_ref=amax_ref.at[pl.ds(d, 1), :],
                send_sem=amax_send_sems.at[off - 1],
                recv_sem=amax_sems.at[d],
                device_id=(tgt,),
                device_id_type=pl.DeviceIdType.MESH,
            )
            sd.start()
            amax_sends.append(sd)
        for off in range(1, N_DEV):
            src = lax.rem(d + off, N_DEV)
            rc = pltpu.make_async_remote_copy(
                src_ref=amax_ref.at[pl.ds(src, 1), :],
                dst_ref=amax_ref.at[pl.ds(src, 1), :],
                send_sem=amax_send_sems.at[off - 1],
                recv_sem=amax_sems.at[src],
                device_id=(d,),
                device_id_type=pl.DeviceIdType.MESH,
            )
            rc.wait_recv()
        for sd in amax_sends:
            sd.wait_send()

        amax = jnp.max(amax_ref[:, :])
        scale = amax / 127.0
        inv = 127.0 / amax
        q = jnp.clip(jnp.round(ychunk_ref[:, :] * inv), -127.0, 127.0)
        comm_ref[0] = (q * scale).astype(jnp.bfloat16)
        cp = pltpu.make_async_copy(
            comm_ref.at[0],
            out_hbm.at[pl.ds(d * CHUNK, CHUNK), :],
            copy_sem,
        )
        cp.start()
        cp.wait()

        for t in range(N_DEV - 1):
            sc = lax.rem(d + (2 * N_DEV - t), N_DEV)
            rdma = pltpu.make_async_remote_copy(
                src_ref=out_hbm.at[pl.ds(sc * CHUNK, CHUNK), :],
                dst_ref=out_hbm.at[pl.ds(sc * CHUNK, CHUNK), :],
                send_sem=send_sems.at[t % 2],
                recv_sem=ag_sems.at[t],
                device_id=(right,),
                device_id_type=pl.DeviceIdType.MESH,
            )
            rdma.start()
            rdma.wait()

    return pl.pallas_call(
        body,
        out_shape=jax.ShapeDtypeStruct((M, N), jnp.bfloat16),
        in_specs=[
            pl.BlockSpec(memory_space=pltpu.VMEM),
            pl.BlockSpec(memory_space=pltpu.VMEM),
        ],
        out_specs=pl.BlockSpec(memory_space=pl.ANY),
        scratch_shapes=[
            pltpu.VMEM((2, CHUNK, N), jnp.bfloat16),
            pltpu.VMEM((CHUNK, N), jnp.float32),
            pltpu.VMEM((N_DEV, 128), jnp.float32),
            pltpu.SemaphoreType.DMA((2,)),
            pltpu.SemaphoreType.DMA((2,)),
            pltpu.SemaphoreType.DMA((N_DEV - 1,)),
            pltpu.SemaphoreType.DMA((N_DEV - 1,)),
            pltpu.SemaphoreType.DMA((N_DEV,)),
            pltpu.SemaphoreType.DMA,
            pltpu.SemaphoreType.REGULAR,
        ],
        compiler_params=pltpu.CompilerParams(collective_id=0),
    )(x, w)


# device time: 1458190 ns/iter; 1.0432x vs baseline; 1.0432x over previous
import jax
import jax.numpy as jnp
from jax import lax
from jax.experimental import pallas as pl
from jax.experimental.pallas import tpu as pltpu

N_DEV = 16
M, K_SH, N = 4096, 256, 8192
CHUNK = M // N_DEV
N2 = N // 2

RING = [0, 1, 2, 3, 4, 5, 6, 9, 10, 11, 12, 13, 14, 15, 8, 7]
POS = [0] * N_DEV
for _r, _g in enumerate(RING):
    POS[_g] = _r


def kernel(x, w_mat):
    x = x.astype(jnp.bfloat16)
    w = w_mat.astype(jnp.bfloat16)

    d = lax.axis_index("i")
    ring = jnp.array(RING, jnp.int32)
    pos = jnp.array(POS, jnp.int32)
    r = pos[d]
    right_g = ring[lax.rem(r + 1, N_DEV)]
    left_g = ring[lax.rem(r + N_DEV - 1, N_DEV)]
    meta = jnp.stack([r, left_g, right_g]).astype(jnp.int32)

    def body(meta_ref, x_ref, w_ref, out_hbm, comm_r, comm_l, ychunk_ref,
             stage_ref, amax_ref, send_r, send_l, recv_r, recv_l,
             ag_r, ag_l, amax_send_sems, amax_sems, copy_sem,
             credit_r, credit_l):
        my_r = meta_ref[0]
        left = meta_ref[1]
        right = meta_ref[2]
        d = lax.axis_index("i")

        barrier_sem = pltpu.get_barrier_semaphore()
        for nbr in (left, right):
            pl.semaphore_signal(
                barrier_sem, inc=1,
                device_id=(nbr,), device_id_type=pl.DeviceIdType.MESH,
            )
        pl.semaphore_wait(barrier_sem, 2)

        def partial_half(c, half):
            xs = x_ref[pl.ds(c * CHUNK, CHUNK), :]
            ws = w_ref[:, pl.ds(half * N2, N2)]
            return jnp.dot(xs, ws, preferred_element_type=jnp.float32)

        for h in range(N_DEV - 1):
            c_r = lax.rem(my_r + (2 * N_DEV - 1 - h), N_DEV)
            c_l = lax.rem(my_r + 1 + h, N_DEV)
            slot = h % 2
            p_r = partial_half(c_r, 0)
            if h == 0:
                comm_r[slot] = p_r.astype(jnp.bfloat16)
            else:
                comm_r[slot] = (comm_r[slot].astype(jnp.float32)
                                + p_r).astype(jnp.bfloat16)
            p_l = partial_half(c_l, 1)
            if h == 0:
                comm_l[slot] = p_l.astype(jnp.bfloat16)
            else:
                comm_l[slot] = (comm_l[slot].astype(jnp.float32)
                                + p_l).astype(jnp.bfloat16)
            if h >= 1:
                pl.semaphore_wait(credit_r, 1)
                pl.semaphore_wait(credit_l, 1)
            rdma_r = pltpu.make_async_remote_copy(
                src_ref=comm_r.at[slot],
                dst_ref=comm_r.at[(h + 1) % 2],
                send_sem=send_r.at[slot],
                recv_sem=recv_r.at[(h + 1) % 2],
                device_id=(right,),
                device_id_type=pl.DeviceIdType.MESH,
            )
            rdma_l = pltpu.make_async_remote_copy(
                src_ref=comm_l.at[slot],
                dst_ref=comm_l.at[(h + 1) % 2],
                send_sem=send_l.at[slot],
                recv_sem=recv_l.at[(h + 1) % 2],
                device_id=(left,),
                device_id_type=pl.DeviceIdType.MESH,
            )
            rdma_r.start()
            rdma_l.start()
            rdma_r.wait()
            rdma_l.wait()
            if h <= N_DEV - 3:
                pl.semaphore_signal(
                    credit_r, inc=1,
                    device_id=(left,), device_id_type=pl.DeviceIdType.MESH,
                )
                pl.semaphore_signal(
                    credit_l, inc=1,
                    device_id=(right,), device_id_type=pl.DeviceIdType.MESH,
                )

        last = (N_DEV - 1) % 2
        ychunk_ref[:, :N2] = comm_r[last].astype(jnp.float32) \
            + partial_half(my_r, 0)
        ychunk_ref[:, N2:] = comm_l[last].astype(jnp.float32) \
            + partial_half(my_r, 1)
        amax_d = jnp.max(jnp.abs(ychunk_ref[:, :]))
        amax_ref[pl.ds(d, 1), :] = jnp.full((1, 128), amax_d, jnp.float32)

        amax_sends = []
        for off in range(1, N_DEV):
            tgt = lax.rem(d + off, N_DEV)
            sd = pltpu.make_async_remote_copy(
                src_ref=amax_ref.at[pl.ds(d, 1), :],
                dst_ref=amax_ref.at[pl.ds(d, 1), :],
                send_sem=amax_send_sems.at[off - 1],
                recv_sem=amax_sems.at[d],
                device_id=(tgt,),
                device_id_type=pl.DeviceIdType.MESH,
            )
            sd.start()
            amax_sends.append(sd)
        for off in range(1, N_DEV):
            src = lax.rem(d + off, N_DEV)
            rc = pltpu.make_async_remote_copy(
                src_ref=amax_ref.at[pl.ds(src, 1), :],
                dst_ref=amax_ref.at[pl.ds(src, 1), :],
                send_sem=amax_send_sems.at[off - 1],
                recv_sem=amax_sems.at[src],
                device_id=(d,),
                device_id_type=pl.DeviceIdType.MESH,
            )
            rc.wait_recv()
        for sd in amax_sends:
            sd.wait_send()

        amax = jnp.max(amax_ref[:, :])
        scale = amax / 127.0
        inv = 127.0 / amax
        q = jnp.clip(jnp.round(ychunk_ref[:, :] * inv), -127.0, 127.0)
        stage_ref[:, :] = (q * scale).astype(jnp.bfloat16)
        cp = pltpu.make_async_copy(
            stage_ref,
            out_hbm.at[pl.ds(my_r * CHUNK, CHUNK), :],
            copy_sem,
        )
        cp.start()
        cp.wait()

        for t in range(N_DEV - 1):
            sc_r = lax.rem(my_r + (2 * N_DEV - t), N_DEV)
            sc_l = lax.rem(my_r + t, N_DEV)
            rdma_r = pltpu.make_async_remote_copy(
                src_ref=out_hbm.at[pl.ds(sc_r * CHUNK, CHUNK), pl.ds(0, N2)],
                dst_ref=out_hbm.at[pl.ds(sc_r * CHUNK, CHUNK), pl.ds(0, N2)],
                send_sem=send_r.at[t % 2],
                recv_sem=ag_r.at[t],
                device_id=(right,),
                device_id_type=pl.DeviceIdType.MESH,
            )
            rdma_l = pltpu.make_async_remote_copy(
                src_ref=out_hbm.at[pl.ds(sc_l * CHUNK, CHUNK), pl.ds(N2, N2)],
                dst_ref=out_hbm.at[pl.ds(sc_l * CHUNK, CHUNK), pl.ds(N2, N2)],
                send_sem=send_l.at[t % 2],
                recv_sem=ag_l.at[t],
                device_id=(left,),
                device_id_type=pl.DeviceIdType.MESH,
            )
            rdma_r.start()
            rdma_l.start()
            rdma_r.wait()
            rdma_l.wait()

    return pl.pallas_call(
        body,
        out_shape=jax.ShapeDtypeStruct((M, N), jnp.bfloat16),
        in_specs=[
            pl.BlockSpec(memory_space=pltpu.MemorySpace.SMEM),
            pl.BlockSpec(memory_space=pltpu.VMEM),
            pl.BlockSpec(memory_space=pltpu.VMEM),
        ],
        out_specs=pl.BlockSpec(memory_space=pl.ANY),
        scratch_shapes=[
            pltpu.VMEM((2, CHUNK, N2), jnp.bfloat16),
            pltpu.VMEM((2, CHUNK, N2), jnp.bfloat16),
            pltpu.VMEM((CHUNK, N), jnp.float32),
            pltpu.VMEM((CHUNK, N), jnp.bfloat16),
            pltpu.VMEM((N_DEV, 128), jnp.float32),
            pltpu.SemaphoreType.DMA((2,)),
            pltpu.SemaphoreType.DMA((2,)),
            pltpu.SemaphoreType.DMA((2,)),
            pltpu.SemaphoreType.DMA((2,)),
            pltpu.SemaphoreType.DMA((N_DEV - 1,)),
            pltpu.SemaphoreType.DMA((N_DEV - 1,)),
            pltpu.SemaphoreType.DMA((N_DEV - 1,)),
            pltpu.SemaphoreType.DMA((N_DEV,)),
            pltpu.SemaphoreType.DMA,
            pltpu.SemaphoreType.REGULAR,
            pltpu.SemaphoreType.REGULAR,
        ],
        compiler_params=pltpu.CompilerParams(collective_id=0),
    )(meta, x, w)
